# baseline (speedup 1.0000x reference)
import functools

import jax
import jax.numpy as jnp
from jax import lax
from jax.experimental import pallas as pl
from jax.experimental.pallas import tpu as pltpu
from jax.experimental.pallas import tpu_sc as plsc

_B, _N, _F = 4, 50000, 100000
_ROW = 16
_DEGCOL = 12
_NC, _NS = 2, 16
_NW = _NC * _NS
_C = 128
_CHUNKS = 25
_FW = _C * _CHUNKS
_FPAD = _FW * _NW
_NPAD = 51200
_RPT = _NPAD // _NS
_TCBLK = 2000


def _sc_body(d16_hbm, fi_hbm, fj_hbm, fk_hbm, out_hbm,
             ii0, ij0, ik0, ii1, ij1, ik1, ii2, ij2, ik2,
             g00, g01, g02, g10, g11, g12,
             s00, s01, s02, s10, s11, s12,
             zbuf, acc, gsem0, gsem1, ssem0, ssem1):
    cid = lax.axis_index("c")
    sid = lax.axis_index("s")
    wid = sid * _NC + cid

    idx = [(ii0, ij0, ik0), (ii1, ij1, ik1), (ii2, ij2, ik2)]
    gbuf = [(g00, g01, g02), (g10, g11, g12)]
    sbuf = [(s00, s01, s02), (s10, s11, s12)]
    gsem = [gsem0, gsem1]
    ssem = [ssem0, ssem1]

    def _zrow(r, carry):
        zbuf[r, :] = jnp.zeros((_ROW,), jnp.float32)
        return carry
    lax.fori_loop(0, _C, _zrow, 0)

    def _zchunk(t, carry):
        pltpu.sync_copy(zbuf, acc.at[pl.ds(sid * _RPT + t * _C, _C)])
        return carry
    lax.fori_loop(0, _CHUNKS, _zchunk, 0)
    plsc.subcore_barrier()

    def _start(t):
        ib = idx[t % 3]
        gb = gbuf[t % 2]
        base = wid * _FW + t * _C
        pltpu.sync_copy(fi_hbm.at[pl.ds(base, _C)], ib[0])
        pltpu.sync_copy(fj_hbm.at[pl.ds(base, _C)], ib[1])
        pltpu.sync_copy(fk_hbm.at[pl.ds(base, _C)], ib[2])
        return [pltpu.async_copy(d16_hbm.at[ib[q]], gb[q], gsem[t % 2])
                for q in range(3)]

    pend = _start(0)
    shandles = [None, None, None]
    for t in range(_CHUNKS):
        b = t % 2
        if shandles[(t - 2) % 3] is not None:
            for h in shandles[(t - 2) % 3]:
                h.wait()
            shandles[(t - 2) % 3] = None
        nxt = _start(t + 1) if t + 1 < _CHUNKS else []
        for h in pend:
            h.wait()
        pend = nxt

        g0, g1, g2 = gbuf[b]
        s0, s1, s2 = sbuf[b]

        def _pair(r, inner):
            a = g0[r, :]
            bb = g1[r, :]
            c = g2[r, :]
            s0[r, :] = bb + c
            s1[r, :] = a + c
            s2[r, :] = a + bb
            return inner
        lax.fori_loop(0, _C, _pair, 0)

        ib = idx[t % 3]
        shandles[t % 3] = [
            pltpu.async_copy(s0, acc.at[ib[0]], ssem[b], add=True),
            pltpu.async_copy(s1, acc.at[ib[1]], ssem[b], add=True),
            pltpu.async_copy(s2, acc.at[ib[2]], ssem[b], add=True),
        ]
    for hs in shandles:
        if hs is not None:
            for h in hs:
                h.wait()

    plsc.subcore_barrier()
    pltpu.sync_copy(acc.at[pl.ds(sid * _RPT, _RPT)],
                    out_hbm.at[cid, pl.ds(sid * _RPT, _RPT)])


_sc_scatter = functools.partial(
    pl.kernel,
    out_type=jax.ShapeDtypeStruct((_NC, _NPAD, _ROW), jnp.float32),
    mesh=plsc.VectorSubcoreMesh(core_axis_name="c", subcore_axis_name="s"),
    compiler_params=pltpu.CompilerParams(use_tc_tiling_on_sc=False),
    scratch_types=(
        [pltpu.VMEM((_C,), jnp.int32)] * 9
        + [pltpu.VMEM((_C, _ROW), jnp.float32)] * 12
        + [pltpu.VMEM((_C, _ROW), jnp.float32)]
        + [pltpu.VMEM_SHARED((_NPAD, _ROW), jnp.float32)]
        + [pltpu.SemaphoreType.DMA] * 4
    ),
)(_sc_body)


_NROWS = _NPAD * _ROW // 128
_RBLK = 1280
_DROWS = _N * 3 // 24


def _prep_body(x_ref, o_ref):
    x = x_ref[...]
    k = lax.broadcasted_iota(jnp.int32, (24, 128), 0)
    l = lax.broadcasted_iota(jnp.int32, (24, 128), 1)
    acc = jnp.zeros((_RBLK, 128), jnp.float32)
    for b in range(_B):
        pb = jnp.where(l == (k // 3) * _ROW + 3 * b + k % 3, 1.0, 0.0)
        acc = acc + jnp.dot(x[b], pb, preferred_element_type=jnp.float32)
    col = lax.broadcasted_iota(jnp.int32, (_RBLK, 128), 1)
    o_ref[...] = acc + jnp.where(col % _ROW == _DEGCOL, 1.0, 0.0)


_tc_prep = pl.pallas_call(
    _prep_body,
    grid=(_NROWS // _RBLK,),
    in_specs=[pl.BlockSpec((_B, _RBLK, 24), lambda i: (0, i, 0))],
    out_specs=pl.BlockSpec((_RBLK, 128), lambda i: (i, 0)),
    out_shape=jax.ShapeDtypeStruct((_NROWS, 128), jnp.float32),
)


def _tc_body(p_ref, d_ref, o_ref, acc_ref):
    i = pl.program_id(0)

    @pl.when(i == 0)
    def _():
        acc_ref[0] = 0.0

    p = p_ref[...]
    s = p[0] + p[1]
    k = lax.broadcasted_iota(jnp.int32, (128, 128), 0)
    l = lax.broadcasted_iota(jnp.int32, (128, 128), 1)
    m = jnp.where(k == (l // _ROW) * _ROW + _DEGCOL, 1.0, 0.0)
    deg = jnp.dot(s, m, preferred_element_type=jnp.float32)
    r = jnp.abs(s / jnp.maximum(deg, 1.0) - d_ref[...])
    col = lax.broadcasted_iota(jnp.int32, (_RBLK, 128), 1)
    row = lax.broadcasted_iota(jnp.int32, (_RBLK, 128), 0) + i * _RBLK
    vid = row * 8 + col // _ROW
    keep = jnp.logical_and(col % _ROW != _DEGCOL, vid < _N)
    acc_ref[0] += jnp.sum(jnp.where(keep, r, 0.0))

    @pl.when(i == pl.num_programs(0) - 1)
    def _():
        o_ref[...] = jnp.full((1, 1), acc_ref[0] * (1.0 / float(_B * _N * 3)),
                              jnp.float32)


_tc_reduce = pl.pallas_call(
    _tc_body,
    grid=(_NROWS // _RBLK,),
    in_specs=[
        pl.BlockSpec((_NC, _RBLK, 128), lambda i: (0, i, 0)),
        pl.BlockSpec((_RBLK, 128), lambda i: (i, 0)),
    ],
    out_specs=pl.BlockSpec((1, 1), lambda i: (0, 0)),
    out_shape=jax.ShapeDtypeStruct((1, 1), jnp.float32),
    scratch_shapes=[pltpu.SMEM((1,), jnp.float32)],
)


@jax.jit
def kernel(vert1, vert2, faces):
    dsub = (vert1 - vert2).reshape(_B, _DROWS, 24)
    d128 = _tc_prep(dsub)
    d16 = d128.reshape(_NPAD, _ROW)
    pad = jnp.full((_FPAD - _F,), _N, jnp.int32)
    fi = jnp.concatenate([faces[:, 0], pad])
    fj = jnp.concatenate([faces[:, 1], pad])
    fk = jnp.concatenate([faces[:, 2], pad])

    partials = _sc_scatter(d16, fi, fj, fk)
    p128 = partials.reshape(_NC, _NROWS, 128)
    out = _tc_reduce(p128, d128)
    return out[0, 0]


# speedup vs baseline: 1.4384x; 1.4384x over previous
import functools

import jax
import jax.numpy as jnp
from jax import lax
from jax.experimental import pallas as pl
from jax.experimental.pallas import tpu as pltpu
from jax.experimental.pallas import tpu_sc as plsc

_B, _N, _F = 4, 50000, 100000
_ROW = 16
_DEGCOL = 12
_NC, _NS = 2, 16
_NW = _NC * _NS
_C = 128
_CHUNKS = 25
_FW = _C * _CHUNKS
_FPAD = _FW * _NW
_NPAD = 51200
_RPT = _NPAD // _NS
_TCBLK = 2000


def _sc_body(d16_hbm, fi_hbm, fj_hbm, fk_hbm, out_hbm,
             ii0, ij0, ik0, ii1, ij1, ik1, ii2, ij2, ik2, ii3, ij3, ik3,
             g00, g01, g02, g10, g11, g12, g20, g21, g22,
             s00, s01, s02, s10, s11, s12,
             zbuf, acc, gsem0, gsem1, gsem2, ssem0, ssem1):
    cid = lax.axis_index("c")
    sid = lax.axis_index("s")
    wid = sid * _NC + cid

    idx = [(ii0, ij0, ik0), (ii1, ij1, ik1), (ii2, ij2, ik2), (ii3, ij3, ik3)]
    gbuf = [(g00, g01, g02), (g10, g11, g12), (g20, g21, g22)]
    sbuf = [(s00, s01, s02), (s10, s11, s12)]
    gsem = [gsem0, gsem1, gsem2]
    ssem = [ssem0, ssem1]

    def _start(t):
        ib = idx[t % 4]
        gb = gbuf[t % 3]
        base = wid * _FW + t * _C
        pltpu.sync_copy(fi_hbm.at[pl.ds(base, _C)], ib[0])
        pltpu.sync_copy(fj_hbm.at[pl.ds(base, _C)], ib[1])
        pltpu.sync_copy(fk_hbm.at[pl.ds(base, _C)], ib[2])
        return [pltpu.async_copy(d16_hbm.at[ib[q]], gb[q], gsem[t % 3])
                for q in range(3)]

    pend = {0: _start(0), 1: _start(1)}

    def _zrow(r, carry):
        zbuf[r, :] = jnp.zeros((_ROW,), jnp.float32)
        return carry
    lax.fori_loop(0, _C, _zrow, 0)

    def _zchunk(t, carry):
        pltpu.sync_copy(zbuf, acc.at[pl.ds(sid * _RPT + t * _C, _C)])
        return carry
    lax.fori_loop(0, _CHUNKS, _zchunk, 0)
    plsc.subcore_barrier()

    shandles = {}
    for t in range(_CHUNKS):
        if t - 2 in shandles:
            for h in shandles.pop(t - 2):
                h.wait()
        if t + 2 < _CHUNKS:
            pend[t + 2] = _start(t + 2)
        for h in pend.pop(t):
            h.wait()

        g0, g1, g2 = gbuf[t % 3]
        s0, s1, s2 = sbuf[t % 2]

        def _pair(r, inner):
            a = g0[r, :]
            bb = g1[r, :]
            c = g2[r, :]
            s0[r, :] = bb + c
            s1[r, :] = a + c
            s2[r, :] = a + bb
            return inner
        lax.fori_loop(0, _C, _pair, 0)

        ib = idx[t % 4]
        shandles[t] = [
            pltpu.async_copy(s0, acc.at[ib[0]], ssem[t % 2], add=True),
            pltpu.async_copy(s1, acc.at[ib[1]], ssem[t % 2], add=True),
            pltpu.async_copy(s2, acc.at[ib[2]], ssem[t % 2], add=True),
        ]
    for hs in shandles.values():
        for h in hs:
            h.wait()

    plsc.subcore_barrier()
    pltpu.sync_copy(acc.at[pl.ds(sid * _RPT, _RPT)],
                    out_hbm.at[cid, pl.ds(sid * _RPT, _RPT)])


_sc_scatter = functools.partial(
    pl.kernel,
    out_type=jax.ShapeDtypeStruct((_NC, _NPAD, _ROW), jnp.float32),
    mesh=plsc.VectorSubcoreMesh(core_axis_name="c", subcore_axis_name="s"),
    compiler_params=pltpu.CompilerParams(use_tc_tiling_on_sc=False),
    scratch_types=(
        [pltpu.VMEM((_C,), jnp.int32)] * 12
        + [pltpu.VMEM((_C, _ROW), jnp.float32)] * 15
        + [pltpu.VMEM((_C, _ROW), jnp.float32)]
        + [pltpu.VMEM_SHARED((_NPAD, _ROW), jnp.float32)]
        + [pltpu.SemaphoreType.DMA] * 5
    ),
)(_sc_body)


_NROWS = _NPAD * _ROW // 128
_RBLK = 1280


def _tc_body(p_ref, d_ref, o_ref, acc_ref):
    i = pl.program_id(0)

    @pl.when(i == 0)
    def _():
        acc_ref[0] = 0.0

    p = p_ref[...]
    s = p[0] + p[1]
    k = lax.broadcasted_iota(jnp.int32, (128, 128), 0)
    l = lax.broadcasted_iota(jnp.int32, (128, 128), 1)
    m = jnp.where(k == (l // _ROW) * _ROW + _DEGCOL, 1.0, 0.0)
    deg = jnp.dot(s, m, preferred_element_type=jnp.float32)
    r = jnp.abs(s / jnp.maximum(deg, 1.0) - d_ref[...])
    col = lax.broadcasted_iota(jnp.int32, (_RBLK, 128), 1)
    row = lax.broadcasted_iota(jnp.int32, (_RBLK, 128), 0) + i * _RBLK
    vid = row * 8 + col // _ROW
    keep = jnp.logical_and(col % _ROW != _DEGCOL, vid < _N)
    acc_ref[0] += jnp.sum(jnp.where(keep, r, 0.0))

    @pl.when(i == pl.num_programs(0) - 1)
    def _():
        o_ref[...] = jnp.full((1, 1), acc_ref[0] * (1.0 / float(_B * _N * 3)),
                              jnp.float32)


_tc_reduce = pl.pallas_call(
    _tc_body,
    grid=(_NROWS // _RBLK,),
    in_specs=[
        pl.BlockSpec((_NC, _RBLK, 128), lambda i: (0, i, 0)),
        pl.BlockSpec((_RBLK, 128), lambda i: (i, 0)),
    ],
    out_specs=pl.BlockSpec((1, 1), lambda i: (0, 0)),
    out_shape=jax.ShapeDtypeStruct((1, 1), jnp.float32),
    scratch_shapes=[pltpu.SMEM((1,), jnp.float32)],
)


@jax.jit
def kernel(vert1, vert2, faces):
    d = vert1 - vert2
    d12 = jnp.transpose(d, (1, 0, 2)).reshape(_N, _B * 3)
    d16 = jnp.concatenate([
        jnp.concatenate([d12,
                         jnp.ones((_N, 1), jnp.float32),
                         jnp.zeros((_N, 3), jnp.float32)], axis=1),
        jnp.zeros((_NPAD - _N, _ROW), jnp.float32),
    ], axis=0)
    d128 = d16.reshape(_NROWS, 128)
    pad = jnp.full((_FPAD - _F,), _N, jnp.int32)
    fi = jnp.concatenate([faces[:, 0], pad])
    fj = jnp.concatenate([faces[:, 1], pad])
    fk = jnp.concatenate([faces[:, 2], pad])

    partials = _sc_scatter(d16, fi, fj, fk)
    p128 = partials.reshape(_NC, _NROWS, 128)
    out = _tc_reduce(p128, d128)
    return out[0, 0]


# speedup vs baseline: 1.4678x; 1.0204x over previous
import functools

import jax
import jax.numpy as jnp
from jax import lax
from jax.experimental import pallas as pl
from jax.experimental.pallas import tpu as pltpu
from jax.experimental.pallas import tpu_sc as plsc

_B, _N, _F = 4, 50000, 100000
_ROW = 16
_DEGCOL = 12
_NC, _NS = 2, 16
_NW = _NC * _NS
_C = 128
_CHUNKS = 25
_FW = _C * _CHUNKS
_FPAD = _FW * _NW
_NPAD = 51200
_RPT = _NPAD // _NS
_TCBLK = 2000


def _sc_body(d16_hbm, fi_hbm, fj_hbm, fk_hbm, out_hbm,
             ii0, ij0, ik0, ii1, ij1, ik1, ii2, ij2, ik2, ii3, ij3, ik3,
             g00, g01, g02, g10, g11, g12, g20, g21, g22, g30, g31, g32,
             zbuf, acc, gsem0, gsem1, gsem2, gsem3, ssem0, ssem1):
    cid = lax.axis_index("c")
    sid = lax.axis_index("s")
    wid = sid * _NC + cid

    idx = [(ii0, ij0, ik0), (ii1, ij1, ik1), (ii2, ij2, ik2), (ii3, ij3, ik3)]
    gbuf = [(g00, g01, g02), (g10, g11, g12), (g20, g21, g22), (g30, g31, g32)]
    gsem = [gsem0, gsem1, gsem2, gsem3]
    ssem = [ssem0, ssem1]

    def _start(t):
        ib = idx[t % 4]
        gb = gbuf[t % 4]
        base = wid * _FW + t * _C
        pltpu.sync_copy(fi_hbm.at[pl.ds(base, _C)], ib[0])
        pltpu.sync_copy(fj_hbm.at[pl.ds(base, _C)], ib[1])
        pltpu.sync_copy(fk_hbm.at[pl.ds(base, _C)], ib[2])
        return [pltpu.async_copy(d16_hbm.at[ib[q]], gb[q], gsem[t % 4])
                for q in range(3)]

    pend = {0: _start(0), 1: _start(1)}

    def _zrow(r, carry):
        zbuf[r, :] = jnp.zeros((_ROW,), jnp.float32)
        return carry
    lax.fori_loop(0, _C, _zrow, 0)

    def _zchunk(t, carry):
        pltpu.sync_copy(zbuf, acc.at[pl.ds(sid * _RPT + t * _C, _C)])
        return carry
    lax.fori_loop(0, _CHUNKS, _zchunk, 0)
    plsc.subcore_barrier()

    shandles = {}
    for t in range(_CHUNKS):
        if t - 2 in shandles:
            for h in shandles.pop(t - 2):
                h.wait()
        if t + 2 < _CHUNKS:
            pend[t + 2] = _start(t + 2)
        for h in pend.pop(t):
            h.wait()

        g0, g1, g2 = gbuf[t % 4]
        ib = idx[t % 4]
        sm = ssem[t % 2]
        shandles[t] = [
            pltpu.async_copy(g1, acc.at[ib[0]], sm, add=True),
            pltpu.async_copy(g2, acc.at[ib[0]], sm, add=True),
            pltpu.async_copy(g0, acc.at[ib[1]], sm, add=True),
            pltpu.async_copy(g2, acc.at[ib[1]], sm, add=True),
            pltpu.async_copy(g0, acc.at[ib[2]], sm, add=True),
            pltpu.async_copy(g1, acc.at[ib[2]], sm, add=True),
        ]
    for hs in shandles.values():
        for h in hs:
            h.wait()

    plsc.subcore_barrier()
    pltpu.sync_copy(acc.at[pl.ds(sid * _RPT, _RPT)],
                    out_hbm.at[cid, pl.ds(sid * _RPT, _RPT)])


_sc_scatter = functools.partial(
    pl.kernel,
    out_type=jax.ShapeDtypeStruct((_NC, _NPAD, _ROW), jnp.float32),
    mesh=plsc.VectorSubcoreMesh(core_axis_name="c", subcore_axis_name="s"),
    compiler_params=pltpu.CompilerParams(use_tc_tiling_on_sc=False),
    scratch_types=(
        [pltpu.VMEM((_C,), jnp.int32)] * 12
        + [pltpu.VMEM((_C, _ROW), jnp.float32)] * 12
        + [pltpu.VMEM((_C, _ROW), jnp.float32)]
        + [pltpu.VMEM_SHARED((_NPAD, _ROW), jnp.float32)]
        + [pltpu.SemaphoreType.DMA] * 6
    ),
)(_sc_body)


_NROWS = _NPAD * _ROW // 128
_RBLK = 1280


def _tc_body(p_ref, d_ref, o_ref, acc_ref):
    i = pl.program_id(0)

    @pl.when(i == 0)
    def _():
        acc_ref[0] = 0.0

    p = p_ref[...]
    s = p[0] + p[1]
    k = lax.broadcasted_iota(jnp.int32, (128, 128), 0)
    l = lax.broadcasted_iota(jnp.int32, (128, 128), 1)
    m = jnp.where(k == (l // _ROW) * _ROW + _DEGCOL, 1.0, 0.0)
    deg = jnp.dot(s, m, preferred_element_type=jnp.float32)
    r = jnp.abs(s / jnp.maximum(deg, 1.0) - d_ref[...])
    col = lax.broadcasted_iota(jnp.int32, (_RBLK, 128), 1)
    row = lax.broadcasted_iota(jnp.int32, (_RBLK, 128), 0) + i * _RBLK
    vid = row * 8 + col // _ROW
    keep = jnp.logical_and(col % _ROW != _DEGCOL, vid < _N)
    acc_ref[0] += jnp.sum(jnp.where(keep, r, 0.0))

    @pl.when(i == pl.num_programs(0) - 1)
    def _():
        o_ref[...] = jnp.full((1, 1), acc_ref[0] * (1.0 / float(_B * _N * 3)),
                              jnp.float32)


_tc_reduce = pl.pallas_call(
    _tc_body,
    grid=(_NROWS // _RBLK,),
    in_specs=[
        pl.BlockSpec((_NC, _RBLK, 128), lambda i: (0, i, 0)),
        pl.BlockSpec((_RBLK, 128), lambda i: (i, 0)),
    ],
    out_specs=pl.BlockSpec((1, 1), lambda i: (0, 0)),
    out_shape=jax.ShapeDtypeStruct((1, 1), jnp.float32),
    scratch_shapes=[pltpu.SMEM((1,), jnp.float32)],
)


@jax.jit
def kernel(vert1, vert2, faces):
    d = vert1 - vert2
    d12 = jnp.transpose(d, (1, 0, 2)).reshape(_N, _B * 3)
    d16 = jnp.concatenate([
        jnp.concatenate([d12,
                         jnp.ones((_N, 1), jnp.float32),
                         jnp.zeros((_N, 3), jnp.float32)], axis=1),
        jnp.zeros((_NPAD - _N, _ROW), jnp.float32),
    ], axis=0)
    d128 = d16.reshape(_NROWS, 128)
    pad = jnp.full((_FPAD - _F,), _N, jnp.int32)
    fi = jnp.concatenate([faces[:, 0], pad])
    fj = jnp.concatenate([faces[:, 1], pad])
    fk = jnp.concatenate([faces[:, 2], pad])

    partials = _sc_scatter(d16, fi, fj, fk)
    p128 = partials.reshape(_NC, _NROWS, 128)
    out = _tc_reduce(p128, d128)
    return out[0, 0]


# speedup vs baseline: 1.5135x; 1.0311x over previous
import functools

import jax
import jax.numpy as jnp
from jax import lax
from jax.experimental import pallas as pl
from jax.experimental.pallas import tpu as pltpu
from jax.experimental.pallas import tpu_sc as plsc

_B, _N, _F = 4, 50000, 100000
_ROW = 16
_DEGCOL = 12
_NC, _NS = 2, 16
_NW = _NC * _NS
_C = 128
_CHUNKS = 25
_FW = _C * _CHUNKS
_FPAD = _FW * _NW
_NPAD = 51200
_RPT = _NPAD // _NS
_TCBLK = 2000


def _sc_body(d16_hbm, f3_hbm, out_hbm,
             ib0, ib1, ib2, ib3,
             g00, g01, g02, g10, g11, g12, g20, g21, g22, g30, g31, g32,
             zbuf, acc, gsem0, gsem1, gsem2, gsem3, ssem0, ssem1):
    cid = lax.axis_index("c")
    sid = lax.axis_index("s")
    wid = sid * _NC + cid

    idx = [ib0, ib1, ib2, ib3]
    gbuf = [(g00, g01, g02), (g10, g11, g12), (g20, g21, g22), (g30, g31, g32)]
    gsem = [gsem0, gsem1, gsem2, gsem3]
    ssem = [ssem0, ssem1]

    def _start(t):
        ib = idx[t % 4]
        gb = gbuf[t % 4]
        pltpu.sync_copy(f3_hbm.at[wid * _CHUNKS + t], ib)
        return [pltpu.async_copy(d16_hbm.at[ib.at[q]], gb[q], gsem[t % 4])
                for q in range(3)]

    pend = {0: _start(0), 1: _start(1)}

    def _zrow(r, carry):
        zbuf[r, :] = jnp.zeros((_ROW,), jnp.float32)
        return carry
    lax.fori_loop(0, _C, _zrow, 0)

    def _zchunk(t, carry):
        pltpu.sync_copy(zbuf, acc.at[pl.ds(sid * _RPT + t * _C, _C)])
        return carry
    lax.fori_loop(0, _CHUNKS, _zchunk, 0)
    plsc.subcore_barrier()

    shandles = {}
    for t in range(_CHUNKS):
        if t - 2 in shandles:
            for h in shandles.pop(t - 2):
                h.wait()
        if t + 2 < _CHUNKS:
            pend[t + 2] = _start(t + 2)
        for h in pend.pop(t):
            h.wait()

        g0, g1, g2 = gbuf[t % 4]
        ib = idx[t % 4]
        sm = ssem[t % 2]
        shandles[t] = [
            pltpu.async_copy(g1, acc.at[ib.at[0]], sm, add=True),
            pltpu.async_copy(g2, acc.at[ib.at[0]], sm, add=True),
            pltpu.async_copy(g0, acc.at[ib.at[1]], sm, add=True),
            pltpu.async_copy(g2, acc.at[ib.at[1]], sm, add=True),
            pltpu.async_copy(g0, acc.at[ib.at[2]], sm, add=True),
            pltpu.async_copy(g1, acc.at[ib.at[2]], sm, add=True),
        ]
    for hs in shandles.values():
        for h in hs:
            h.wait()

    plsc.subcore_barrier()
    pltpu.sync_copy(acc.at[pl.ds(sid * _RPT, _RPT)],
                    out_hbm.at[cid, pl.ds(sid * _RPT, _RPT)])


_sc_scatter = functools.partial(
    pl.kernel,
    out_type=jax.ShapeDtypeStruct((_NC, _NPAD, _ROW), jnp.float32),
    mesh=plsc.VectorSubcoreMesh(core_axis_name="c", subcore_axis_name="s"),
    compiler_params=pltpu.CompilerParams(use_tc_tiling_on_sc=False),
    scratch_types=(
        [pltpu.VMEM((3, _C), jnp.int32)] * 4
        + [pltpu.VMEM((_C, _ROW), jnp.float32)] * 12
        + [pltpu.VMEM((_C, _ROW), jnp.float32)]
        + [pltpu.VMEM_SHARED((_NPAD, _ROW), jnp.float32)]
        + [pltpu.SemaphoreType.DMA] * 6
    ),
)(_sc_body)


_NROWS = _NPAD * _ROW // 128
_RBLK = 1280


def _tc_body(p_ref, d_ref, o_ref, acc_ref):
    i = pl.program_id(0)

    @pl.when(i == 0)
    def _():
        acc_ref[0] = 0.0

    p = p_ref[...]
    s = p[0] + p[1]
    k = lax.broadcasted_iota(jnp.int32, (128, 128), 0)
    l = lax.broadcasted_iota(jnp.int32, (128, 128), 1)
    m = jnp.where(k == (l // _ROW) * _ROW + _DEGCOL, 1.0, 0.0)
    deg = jnp.dot(s, m, preferred_element_type=jnp.float32)
    r = jnp.abs(s / jnp.maximum(deg, 1.0) - d_ref[...])
    col = lax.broadcasted_iota(jnp.int32, (_RBLK, 128), 1)
    row = lax.broadcasted_iota(jnp.int32, (_RBLK, 128), 0) + i * _RBLK
    vid = row * 8 + col // _ROW
    keep = jnp.logical_and(col % _ROW != _DEGCOL, vid < _N)
    acc_ref[0] += jnp.sum(jnp.where(keep, r, 0.0))

    @pl.when(i == pl.num_programs(0) - 1)
    def _():
        o_ref[...] = jnp.full((1, 1), acc_ref[0] * (1.0 / float(_B * _N * 3)),
                              jnp.float32)


_tc_reduce = pl.pallas_call(
    _tc_body,
    grid=(_NROWS // _RBLK,),
    in_specs=[
        pl.BlockSpec((_NC, _RBLK, 128), lambda i: (0, i, 0)),
        pl.BlockSpec((_RBLK, 128), lambda i: (i, 0)),
    ],
    out_specs=pl.BlockSpec((1, 1), lambda i: (0, 0)),
    out_shape=jax.ShapeDtypeStruct((1, 1), jnp.float32),
    scratch_shapes=[pltpu.SMEM((1,), jnp.float32)],
)


@jax.jit
def kernel(vert1, vert2, faces):
    d = vert1 - vert2
    d12 = jnp.transpose(d, (1, 0, 2)).reshape(_N, _B * 3)
    d16 = jnp.concatenate([
        jnp.concatenate([d12,
                         jnp.ones((_N, 1), jnp.float32),
                         jnp.zeros((_N, 3), jnp.float32)], axis=1),
        jnp.zeros((_NPAD - _N, _ROW), jnp.float32),
    ], axis=0)
    d128 = d16.reshape(_NROWS, 128)
    pad = jnp.full((_FPAD - _F,), _N, jnp.int32)
    fr = [jnp.concatenate([faces[:, q], pad]).reshape(_NW, _CHUNKS, _C)
          for q in range(3)]
    f3 = jnp.stack(fr, axis=2).reshape(_NW * _CHUNKS, 3, _C)

    partials = _sc_scatter(d16, f3)
    p128 = partials.reshape(_NC, _NROWS, 128)
    out = _tc_reduce(p128, d128)
    return out[0, 0]


# speedup vs baseline: 1.5167x; 1.0021x over previous
import functools

import jax
import jax.numpy as jnp
from jax import lax
from jax.experimental import pallas as pl
from jax.experimental.pallas import tpu as pltpu
from jax.experimental.pallas import tpu_sc as plsc

_B, _N, _F = 4, 50000, 100000
_ROW = 16
_DEGCOL = 12
_NC, _NS = 2, 16
_NW = _NC * _NS
_C = 128
_CHUNKS = 25
_FW = _C * _CHUNKS
_FPAD = _FW * _NW
_NPAD = 51200
_RPT = _NPAD // _NS
_TCBLK = 2000


_DEPTH = 3
_NBUF = 5


def _sc_body(d16_hbm, f3_hbm, out_hbm,
             ib0, ib1, ib2, ib3, ib4,
             g00, g01, g02, g10, g11, g12, g20, g21, g22, g30, g31, g32,
             g40, g41, g42,
             zbuf, acc, gsem0, gsem1, gsem2, gsem3, gsem4, ssem0, ssem1):
    cid = lax.axis_index("c")
    sid = lax.axis_index("s")
    wid = sid * _NC + cid

    idx = [ib0, ib1, ib2, ib3, ib4]
    gbuf = [(g00, g01, g02), (g10, g11, g12), (g20, g21, g22),
            (g30, g31, g32), (g40, g41, g42)]
    gsem = [gsem0, gsem1, gsem2, gsem3, gsem4]
    ssem = [ssem0, ssem1]

    def _start(t):
        ib = idx[t % _NBUF]
        gb = gbuf[t % _NBUF]
        pltpu.sync_copy(f3_hbm.at[wid * _CHUNKS + t], ib)
        return [pltpu.async_copy(d16_hbm.at[ib.at[q]], gb[q], gsem[t % _NBUF])
                for q in range(3)]

    pend = {t: _start(t) for t in range(_DEPTH)}

    def _zrow(r, carry):
        zbuf[r, :] = jnp.zeros((_ROW,), jnp.float32)
        return carry
    lax.fori_loop(0, _C, _zrow, 0)

    def _zchunk(t, carry):
        pltpu.sync_copy(zbuf, acc.at[pl.ds(sid * _RPT + t * _C, _C)])
        return carry
    lax.fori_loop(0, _CHUNKS, _zchunk, 0)
    plsc.subcore_barrier()

    shandles = {}
    for t in range(_CHUNKS):
        if t - 2 in shandles:
            for h in shandles.pop(t - 2):
                h.wait()
        if t + _DEPTH < _CHUNKS:
            pend[t + _DEPTH] = _start(t + _DEPTH)
        for h in pend.pop(t):
            h.wait()

        g0, g1, g2 = gbuf[t % _NBUF]
        ib = idx[t % _NBUF]
        sm = ssem[t % 2]
        shandles[t] = [
            pltpu.async_copy(g1, acc.at[ib.at[0]], sm, add=True),
            pltpu.async_copy(g2, acc.at[ib.at[0]], sm, add=True),
            pltpu.async_copy(g0, acc.at[ib.at[1]], sm, add=True),
            pltpu.async_copy(g2, acc.at[ib.at[1]], sm, add=True),
            pltpu.async_copy(g0, acc.at[ib.at[2]], sm, add=True),
            pltpu.async_copy(g1, acc.at[ib.at[2]], sm, add=True),
        ]
    for hs in shandles.values():
        for h in hs:
            h.wait()

    plsc.subcore_barrier()
    pltpu.sync_copy(acc.at[pl.ds(sid * _RPT, _RPT)],
                    out_hbm.at[cid, pl.ds(sid * _RPT, _RPT)])


_sc_scatter = functools.partial(
    pl.kernel,
    out_type=jax.ShapeDtypeStruct((_NC, _NPAD, _ROW), jnp.float32),
    mesh=plsc.VectorSubcoreMesh(core_axis_name="c", subcore_axis_name="s"),
    compiler_params=pltpu.CompilerParams(use_tc_tiling_on_sc=False),
    scratch_types=(
        [pltpu.VMEM((3, _C), jnp.int32)] * 5
        + [pltpu.VMEM((_C, _ROW), jnp.float32)] * 15
        + [pltpu.VMEM((_C, _ROW), jnp.float32)]
        + [pltpu.VMEM_SHARED((_NPAD, _ROW), jnp.float32)]
        + [pltpu.SemaphoreType.DMA] * 7
    ),
)(_sc_body)


_NROWS = _NPAD * _ROW // 128
_RBLK = 1280


def _tc_body(p_ref, d_ref, o_ref, acc_ref):
    i = pl.program_id(0)

    @pl.when(i == 0)
    def _():
        acc_ref[0] = 0.0

    p = p_ref[...]
    s = p[0] + p[1]
    k = lax.broadcasted_iota(jnp.int32, (128, 128), 0)
    l = lax.broadcasted_iota(jnp.int32, (128, 128), 1)
    m = jnp.where(k == (l // _ROW) * _ROW + _DEGCOL, 1.0, 0.0)
    deg = jnp.dot(s, m, preferred_element_type=jnp.float32)
    r = jnp.abs(s / jnp.maximum(deg, 1.0) - d_ref[...])
    col = lax.broadcasted_iota(jnp.int32, (_RBLK, 128), 1)
    row = lax.broadcasted_iota(jnp.int32, (_RBLK, 128), 0) + i * _RBLK
    vid = row * 8 + col // _ROW
    keep = jnp.logical_and(col % _ROW != _DEGCOL, vid < _N)
    acc_ref[0] += jnp.sum(jnp.where(keep, r, 0.0))

    @pl.when(i == pl.num_programs(0) - 1)
    def _():
        o_ref[...] = jnp.full((1, 1), acc_ref[0] * (1.0 / float(_B * _N * 3)),
                              jnp.float32)


_tc_reduce = pl.pallas_call(
    _tc_body,
    grid=(_NROWS // _RBLK,),
    in_specs=[
        pl.BlockSpec((_NC, _RBLK, 128), lambda i: (0, i, 0)),
        pl.BlockSpec((_RBLK, 128), lambda i: (i, 0)),
    ],
    out_specs=pl.BlockSpec((1, 1), lambda i: (0, 0)),
    out_shape=jax.ShapeDtypeStruct((1, 1), jnp.float32),
    scratch_shapes=[pltpu.SMEM((1,), jnp.float32)],
)


@jax.jit
def kernel(vert1, vert2, faces):
    d = vert1 - vert2
    d12 = jnp.transpose(d, (1, 0, 2)).reshape(_N, _B * 3)
    d16 = jnp.concatenate([
        jnp.concatenate([d12,
                         jnp.ones((_N, 1), jnp.float32),
                         jnp.zeros((_N, 3), jnp.float32)], axis=1),
        jnp.zeros((_NPAD - _N, _ROW), jnp.float32),
    ], axis=0)
    d128 = d16.reshape(_NROWS, 128)
    pad = jnp.full((_FPAD - _F,), _N, jnp.int32)
    fr = [jnp.concatenate([faces[:, q], pad]).reshape(_NW, _CHUNKS, _C)
          for q in range(3)]
    f3 = jnp.stack(fr, axis=2).reshape(_NW * _CHUNKS, 3, _C)

    partials = _sc_scatter(d16, f3)
    p128 = partials.reshape(_NC, _NROWS, 128)
    out = _tc_reduce(p128, d128)
    return out[0, 0]
